# baseline (speedup 1.0000x reference)
import functools

import jax
import jax.numpy as jnp
import numpy as np
from jax import lax
from jax.experimental import pallas as pl
from jax.experimental.pallas import tpu as pltpu
from jax.experimental.pallas import tpu_sc as plsc

DMODEL = 1024
N_EXPERTS = 16
EXPERT_SIZE = 128
N_HEADS = 2
EH = N_EXPERTS * EXPERT_SIZE
N_TOK = 4096

TB = 512

_info = plsc.get_sparse_core_info()
_NC, _NS, _NL = _info.num_cores, _info.num_subcores, _info.num_lanes
_NW = _NC * _NS
_TPW = N_TOK // _NW


def _sel_block(x_ref, es_ref, selt_ref):
    selt_ref[...] = lax.dot_general(
        es_ref[...], x_ref[...],
        dimension_numbers=(((1,), (1,)), ((), ())),
        preferred_element_type=jnp.float32,
    )


def _router_logits_t(x, expert_sel):
    blk = 1024
    return pl.pallas_call(
        _sel_block,
        grid=(N_TOK // blk,),
        in_specs=[
            pl.BlockSpec((blk, DMODEL), lambda i: (i, 0)),
            pl.BlockSpec((N_EXPERTS, DMODEL), lambda i: (0, 0)),
        ],
        out_specs=pl.BlockSpec((N_EXPERTS, blk), lambda i: (0, i)),
        out_shape=jax.ShapeDtypeStruct((N_EXPERTS, N_TOK), jnp.float32),
    )(x, expert_sel)


@functools.partial(
    pl.kernel,
    mesh=plsc.VectorSubcoreMesh(core_axis_name="c", subcore_axis_name="s"),
    out_type=jax.ShapeDtypeStruct((N_EXPERTS, N_TOK), jnp.float32),
    scratch_types=[
        pltpu.VMEM((N_EXPERTS, _TPW), jnp.float32),
        pltpu.VMEM((N_EXPERTS, _TPW), jnp.float32),
    ],
)
def _route_sc(selt_hbm, gt_hbm, sel_v, g_v):
    wid = lax.axis_index("s") * _NC + lax.axis_index("c")
    base = wid * _TPW
    pltpu.sync_copy(selt_hbm.at[:, pl.ds(base, _TPW)], sel_v)

    neg_big = jnp.full((_NL,), -3.0e38, jnp.float32)
    neg_gate = jnp.full((_NL,), -1.0e30, jnp.float32)

    for j in range(_TPW // _NL):
        sl = pl.ds(j * _NL, _NL)
        v = [sel_v[e, sl] for e in range(N_EXPERTS)]
        def tree_max(vals):
            while len(vals) > 1:
                vals = [jnp.maximum(vals[i], vals[i + 1])
                        for i in range(0, len(vals), 2)]
            return vals[0]
        m1 = tree_max(v)
        v2 = [jnp.where(v[e] == m1, neg_big, v[e]) for e in range(N_EXPERTS)]
        m2 = tree_max(v2)
        for e in range(N_EXPERTS):
            keep = (v[e] == m1) | (v[e] == m2)
            g_v[e, sl] = jnp.where(keep, v[e], neg_gate)

    pltpu.sync_copy(g_v, gt_hbm.at[:, pl.ds(base, _TPW)])


def _moe_block(xb_ref, gt_ref, ex_ref, k3_ref, v_ref, out_ref, k_scr):
    @pl.when(pl.program_id(0) == 0)
    def _relayout_keys():
        for e in range(N_EXPERTS):
            k_scr[:, e * EXPERT_SIZE:(e + 1) * EXPERT_SIZE] = k3_ref[e]

    g_big = lax.dot_general(
        gt_ref[...], ex_ref[...],
        dimension_numbers=(((0,), (0,)), ((), ())),
        preferred_element_type=jnp.float32,
    )
    h = jnp.dot(xb_ref[...], k_scr[...], preferred_element_type=jnp.float32)
    h = jnp.maximum(h + g_big, 0.0)
    out_ref[...] = jnp.dot(h.astype(jnp.bfloat16), v_ref[...],
                           preferred_element_type=jnp.float32)


def _moe_dense(xb, g_t, expand, k3, v_all):
    return pl.pallas_call(
        _moe_block,
        grid=(N_TOK // TB,),
        in_specs=[
            pl.BlockSpec((TB, DMODEL), lambda i: (i, 0)),
            pl.BlockSpec((N_EXPERTS, TB), lambda i: (0, i)),
            pl.BlockSpec((N_EXPERTS, EH), lambda i: (0, 0)),
            pl.BlockSpec((N_EXPERTS, DMODEL, EXPERT_SIZE), lambda i: (0, 0, 0)),
            pl.BlockSpec((EH, DMODEL), lambda i: (0, 0)),
        ],
        out_specs=pl.BlockSpec((TB, DMODEL), lambda i: (i, 0)),
        out_shape=jax.ShapeDtypeStruct((N_TOK, DMODEL), jnp.float32),
        scratch_shapes=[pltpu.VMEM((DMODEL, EH), jnp.bfloat16)],
    )(xb, g_t, expand, k3, v_all)


@jax.jit
def kernel(x, keys_w, values, expert_sel):
    xb = x.astype(jnp.bfloat16)
    k3 = keys_w.astype(jnp.bfloat16)
    v_all = values.reshape(EH, DMODEL).astype(jnp.bfloat16)
    expand = jnp.asarray(
        np.kron(np.eye(N_EXPERTS, dtype=np.float32),
                np.ones((1, EXPERT_SIZE), dtype=np.float32)))

    sel_t = _router_logits_t(x, expert_sel)
    g_t = _route_sc(sel_t)
    return _moe_dense(xb, g_t, expand, k3, v_all)


# speedup vs baseline: 1.1197x; 1.1197x over previous
import functools

import jax
import jax.numpy as jnp
import numpy as np
from jax import lax
from jax.experimental import pallas as pl
from jax.experimental.pallas import tpu as pltpu
from jax.experimental.pallas import tpu_sc as plsc

DMODEL = 1024
N_EXPERTS = 16
EXPERT_SIZE = 128
N_HEADS = 2
EH = N_EXPERTS * EXPERT_SIZE
N_TOK = 4096

TB = 512

_info = plsc.get_sparse_core_info()
_NC, _NS, _NL = _info.num_cores, _info.num_subcores, _info.num_lanes
_NW = _NC * _NS
_TPW = N_TOK // _NW


def _sel_block(x_ref, es_ref, selt_ref):
    selt_ref[...] = lax.dot_general(
        es_ref[...], x_ref[...],
        dimension_numbers=(((1,), (1,)), ((), ())),
        preferred_element_type=jnp.float32,
    )


def _router_logits_t(x, expert_sel):
    blk = 1024
    return pl.pallas_call(
        _sel_block,
        grid=(N_TOK // blk,),
        in_specs=[
            pl.BlockSpec((blk, DMODEL), lambda i: (i, 0)),
            pl.BlockSpec((N_EXPERTS, DMODEL), lambda i: (0, 0)),
        ],
        out_specs=pl.BlockSpec((N_EXPERTS, blk), lambda i: (0, i)),
        out_shape=jax.ShapeDtypeStruct((N_EXPERTS, N_TOK), jnp.float32),
    )(x, expert_sel)


@functools.partial(
    pl.kernel,
    mesh=plsc.VectorSubcoreMesh(core_axis_name="c", subcore_axis_name="s"),
    out_type=jax.ShapeDtypeStruct((N_EXPERTS, N_TOK), jnp.float32),
    scratch_types=[
        pltpu.VMEM((N_EXPERTS, _TPW), jnp.float32),
        pltpu.VMEM((N_EXPERTS, _TPW), jnp.float32),
    ],
)
def _route_sc(selt_hbm, gt_hbm, sel_v, g_v):
    wid = lax.axis_index("s") * _NC + lax.axis_index("c")
    base = wid * _TPW
    pltpu.sync_copy(selt_hbm.at[:, pl.ds(base, _TPW)], sel_v)

    neg_big = jnp.full((_NL,), -3.0e38, jnp.float32)
    neg_gate = jnp.full((_NL,), -1.0e30, jnp.float32)

    for j in range(_TPW // _NL):
        sl = pl.ds(j * _NL, _NL)
        v = [sel_v[e, sl] for e in range(N_EXPERTS)]
        def tree_max(vals):
            while len(vals) > 1:
                vals = [jnp.maximum(vals[i], vals[i + 1])
                        for i in range(0, len(vals), 2)]
            return vals[0]
        m1 = tree_max(v)
        v2 = [jnp.where(v[e] == m1, neg_big, v[e]) for e in range(N_EXPERTS)]
        m2 = tree_max(v2)
        for e in range(N_EXPERTS):
            keep = (v[e] == m1) | (v[e] == m2)
            g_v[e, sl] = jnp.where(keep, v[e], neg_gate)

    pltpu.sync_copy(g_v, gt_hbm.at[:, pl.ds(base, _TPW)])


def _moe_block(xb_ref, gt_ref, ex_ref, k3_ref, v_ref, out_ref, k_scr):
    @pl.when(pl.program_id(0) == 0)
    def _relayout_keys():
        for e in range(N_EXPERTS):
            k_scr[:, e * EXPERT_SIZE:(e + 1) * EXPERT_SIZE] = k3_ref[e]

    g_big = lax.dot_general(
        gt_ref[...], ex_ref[...],
        dimension_numbers=(((0,), (0,)), ((), ())),
        preferred_element_type=jnp.float32,
    )
    h = jnp.dot(xb_ref[...].astype(jnp.bfloat16), k_scr[...],
                preferred_element_type=jnp.float32)
    h = jnp.maximum(h + g_big, 0.0)
    out_ref[...] = jnp.dot(h.astype(jnp.bfloat16), v_ref[...],
                           preferred_element_type=jnp.float32)


def _moe_dense(xb, g_t, expand, k3, v_all):
    return pl.pallas_call(
        _moe_block,
        grid=(N_TOK // TB,),
        in_specs=[
            pl.BlockSpec((TB, DMODEL), lambda i: (i, 0)),
            pl.BlockSpec((N_EXPERTS, TB), lambda i: (0, i)),
            pl.BlockSpec((N_EXPERTS, EH), lambda i: (0, 0)),
            pl.BlockSpec((N_EXPERTS, DMODEL, EXPERT_SIZE), lambda i: (0, 0, 0)),
            pl.BlockSpec((EH, DMODEL), lambda i: (0, 0)),
        ],
        out_specs=pl.BlockSpec((TB, DMODEL), lambda i: (i, 0)),
        out_shape=jax.ShapeDtypeStruct((N_TOK, DMODEL), jnp.float32),
        scratch_shapes=[pltpu.VMEM((DMODEL, EH), jnp.bfloat16)],
    )(xb, g_t, expand, k3, v_all)


@jax.jit
def kernel(x, keys_w, values, expert_sel):
    k3 = keys_w.astype(jnp.bfloat16)
    v_all = values.reshape(EH, DMODEL).astype(jnp.bfloat16)
    expand = jnp.asarray(
        np.kron(np.eye(N_EXPERTS, dtype=np.float32),
                np.ones((1, EXPERT_SIZE), dtype=np.float32)))

    sel_t = _router_logits_t(x, expert_sel)
    g_t = _route_sc(sel_t)
    return _moe_dense(x, g_t, expand, k3, v_all)


# speedup vs baseline: 1.1225x; 1.0025x over previous
import functools

import jax
import jax.numpy as jnp
import numpy as np
from jax import lax
from jax.experimental import pallas as pl
from jax.experimental.pallas import tpu as pltpu
from jax.experimental.pallas import tpu_sc as plsc

DMODEL = 1024
N_EXPERTS = 16
EXPERT_SIZE = 128
N_HEADS = 2
EH = N_EXPERTS * EXPERT_SIZE
N_TOK = 4096

TB = 512

_info = plsc.get_sparse_core_info()
_NC, _NS, _NL = _info.num_cores, _info.num_subcores, _info.num_lanes
_NW = _NC * _NS
_TPW = N_TOK // _NW


def _sel_block(x_ref, es_ref, selt_ref):
    selt_ref[...] = lax.dot_general(
        es_ref[...], x_ref[...],
        dimension_numbers=(((1,), (1,)), ((), ())),
        preferred_element_type=jnp.float32,
    )


def _router_logits_t(x, expert_sel):
    blk = 1024
    return pl.pallas_call(
        _sel_block,
        grid=(N_TOK // blk,),
        in_specs=[
            pl.BlockSpec((blk, DMODEL), lambda i: (i, 0)),
            pl.BlockSpec((N_EXPERTS, DMODEL), lambda i: (0, 0)),
        ],
        out_specs=pl.BlockSpec((N_EXPERTS, blk), lambda i: (0, i)),
        out_shape=jax.ShapeDtypeStruct((N_EXPERTS, N_TOK), jnp.float32),
    )(x, expert_sel)


@functools.partial(
    pl.kernel,
    mesh=plsc.VectorSubcoreMesh(core_axis_name="c", subcore_axis_name="s"),
    out_type=jax.ShapeDtypeStruct((N_EXPERTS, N_TOK), jnp.float32),
    scratch_types=[
        pltpu.VMEM((N_EXPERTS, _TPW), jnp.float32),
        pltpu.VMEM((N_EXPERTS, _TPW), jnp.float32),
    ],
)
def _route_sc(selt_hbm, gt_hbm, sel_v, g_v):
    wid = lax.axis_index("s") * _NC + lax.axis_index("c")
    base = wid * _TPW
    pltpu.sync_copy(selt_hbm.at[:, pl.ds(base, _TPW)], sel_v)

    neg_big = jnp.full((_NL,), -3.0e38, jnp.float32)
    neg_gate = jnp.full((_NL,), -1.0e30, jnp.float32)

    for j in range(_TPW // _NL):
        sl = pl.ds(j * _NL, _NL)
        v = [sel_v[e, sl] for e in range(N_EXPERTS)]
        def tree_max(vals):
            while len(vals) > 1:
                vals = [jnp.maximum(vals[i], vals[i + 1])
                        for i in range(0, len(vals), 2)]
            return vals[0]
        m1 = tree_max(v)
        v2 = [jnp.where(v[e] == m1, neg_big, v[e]) for e in range(N_EXPERTS)]
        m2 = tree_max(v2)
        for e in range(N_EXPERTS):
            keep = (v[e] == m1) | (v[e] == m2)
            g_v[e, sl] = jnp.where(keep, v[e], neg_gate)

    pltpu.sync_copy(g_v, gt_hbm.at[:, pl.ds(base, _TPW)])


def _moe_block(xb_ref, gt_ref, ex_ref, k3_ref, v_ref, out_ref, k_scr):
    @pl.when(pl.program_id(0) == 0)
    def _relayout_keys():
        for e in range(N_EXPERTS):
            k_scr[:, e * EXPERT_SIZE:(e + 1) * EXPERT_SIZE] = k3_ref[e]

    g_big = lax.dot_general(
        gt_ref[...].astype(jnp.bfloat16), ex_ref[...],
        dimension_numbers=(((0,), (0,)), ((), ())),
        preferred_element_type=jnp.float32,
    )
    h = jnp.dot(xb_ref[...].astype(jnp.bfloat16), k_scr[...],
                preferred_element_type=jnp.float32)
    h = jnp.maximum(h + g_big, 0.0)
    out_ref[...] = jnp.dot(h.astype(jnp.bfloat16), v_ref[...],
                           preferred_element_type=jnp.float32)


def _moe_dense(xb, g_t, expand, k3, v_all):
    return pl.pallas_call(
        _moe_block,
        grid=(N_TOK // TB,),
        in_specs=[
            pl.BlockSpec((TB, DMODEL), lambda i: (i, 0)),
            pl.BlockSpec((N_EXPERTS, TB), lambda i: (0, i)),
            pl.BlockSpec((N_EXPERTS, EH), lambda i: (0, 0)),
            pl.BlockSpec((N_EXPERTS, DMODEL, EXPERT_SIZE), lambda i: (0, 0, 0)),
            pl.BlockSpec((EH, DMODEL), lambda i: (0, 0)),
        ],
        out_specs=pl.BlockSpec((TB, DMODEL), lambda i: (i, 0)),
        out_shape=jax.ShapeDtypeStruct((N_TOK, DMODEL), jnp.float32),
        scratch_shapes=[pltpu.VMEM((DMODEL, EH), jnp.bfloat16)],
    )(xb, g_t, expand, k3, v_all)


@jax.jit
def kernel(x, keys_w, values, expert_sel):
    k3 = keys_w.astype(jnp.bfloat16)
    v_all = values.reshape(EH, DMODEL).astype(jnp.bfloat16)
    expand = jnp.asarray(
        np.kron(np.eye(N_EXPERTS, dtype=np.float32),
                np.ones((1, EXPERT_SIZE), dtype=np.float32)),
        dtype=jnp.bfloat16)

    sel_t = _router_logits_t(x, expert_sel)
    g_t = _route_sc(sel_t)
    return _moe_dense(x, g_t, expand, k3, v_all)
